# baseline (speedup 1.0000x reference)
import functools

import jax
import jax.numpy as jnp
from jax import lax
from jax.experimental import pallas as pl
from jax.experimental.pallas import tpu as pltpu
from jax.experimental.pallas import tpu_sc as plsc

_L = 16


def _make_sc_lookup(b: int, t: int, v: int, d: int):
  info = plsc.get_sparse_core_info()
  nw = info.num_cores * info.num_subcores
  assert v == 32 and b % 128 == 0
  nvt = v // 8
  nbt = b // 128
  n_units = t * nvt
  assert n_units % nw == 0
  per_w = n_units // nw

  mesh = plsc.VectorSubcoreMesh(core_axis_name="c", subcore_axis_name="s")

  @functools.partial(
      pl.kernel,
      out_type=jax.ShapeDtypeStruct((t, nvt, nbt, 8, 128), jnp.float32),
      mesh=mesh,
      compiler_params=pltpu.CompilerParams(
          needs_layout_passes=False, use_tc_tiling_on_sc=False
      ),
      scratch_types=[
          pltpu.VMEM((v, d), jnp.float32),
          pltpu.VMEM((v, d), jnp.float32),
          pltpu.VMEM((1, v), jnp.float32),
          pltpu.VMEM((v * v * _L,), jnp.float32),
          pltpu.VMEM((2, b), jnp.int32),
          pltpu.VMEM((2, nbt, 8, 128), jnp.float32),
          pltpu.SemaphoreType.DMA,
          pltpu.SemaphoreType.DMA,
          pltpu.SemaphoreType.DMA,
          pltpu.SemaphoreType.DMA,
      ],
  )
  def sc_lookup(ids_hbm, emb_hbm, head_hbm, bias_hbm, out_hbm,
                emb_v, head_v, bias_v, table_t, ids_vv, out_vv,
                sem_i0, sem_i1, sem_o0, sem_o1):
    cid = lax.axis_index("c")
    sid = lax.axis_index("s")
    wid = sid * info.num_cores + cid

    iota = lax.iota(jnp.int32, _L)

    def build(p, _):
      p16 = pl.multiple_of(p * _L, _L)
      pvec = p16 + iota
      ivec = lax.shift_right_logical(pvec, 5)
      jvec = lax.bitwise_and(pvec, v - 1)
      zvec = jnp.zeros((_L,), jnp.int32)
      acc = plsc.load_gather(bias_v, [zvec, jvec])
      for dd in range(d):
        dvec = jnp.full((_L,), dd, jnp.int32)
        e = plsc.load_gather(emb_v, [ivec, dvec])
        h = plsc.load_gather(head_v, [jvec, dvec])
        acc = acc + e * h
      bvec = lax.shift_left(jvec, 9) + lax.shift_left(ivec, 4)
      for m in range(_L):
        rot = lax.bitwise_and(iota + m, _L - 1)
        plsc.store_scatter(table_t, [bvec + rot], acc)
      return _

    sem_ids = (sem_i0, sem_i1)
    sem_out = (sem_o0, sem_o1)

    def unit_tv(j):
      u = wid * per_w + j
      return u // nvt, lax.rem(u, nvt)

    def ids_copy(j, buf):
      tt, _ = unit_tv(j)
      return pltpu.make_async_copy(
          ids_hbm.at[tt], ids_vv.at[buf], sem_ids[buf])

    ids_copy(0, 0).start()
    pltpu.sync_copy(emb_hbm, emb_v)
    pltpu.sync_copy(head_hbm, head_v)
    pltpu.sync_copy(bias_hbm, bias_v.at[0])
    lax.fori_loop(0, (v * v) // _L, build, 0)

    def out_copy(j, buf):
      tt, vt = unit_tv(j)
      return pltpu.make_async_copy(
          out_vv.at[buf], out_hbm.at[tt, vt], sem_out[buf])

    def compute(j, bufsel):
      _, vt = unit_tv(j)
      cb512 = (vt * 8) * (v * _L) + jnp.zeros((_L,), jnp.int32)

      @plsc.parallel_loop(0, nbt, 1)
      def bt_body(bt):
        bases = []
        for bi0 in range(8):
          off = pl.multiple_of(bt * 128 + bi0 * _L, _L)
          idvec = ids_vv[bufsel, pl.ds(off, _L)]
          bases.append(
              lax.bitwise_or(lax.shift_left(idvec, 4), iota) + cb512)
        for vi in range(8):
          for bi0 in range(8):
            val = plsc.load_gather(table_t, [bases[bi0] + vi * (v * _L)])
            out_vv[bufsel, bt, vi, pl.ds(bi0 * _L, _L)] = val

    assert per_w >= 4 and per_w % 2 == 1
    n_pairs = per_w // 2

    ids_copy(0, 0).wait()
    ids_copy(1, 1).start()
    compute(0, 0)
    out_copy(0, 0).start()
    ids_copy(1, 1).wait()
    ids_copy(2, 0).start()
    compute(1, 1)
    out_copy(1, 1).start()

    def do_pair(k, _):
      j0 = k * 2
      ids_copy(j0, 0).wait()
      ids_copy(j0 + 1, 1).start()
      out_copy(j0 - 2, 0).wait()
      compute(j0, 0)
      out_copy(j0, 0).start()
      ids_copy(j0 + 1, 1).wait()
      ids_copy(j0 + 2, 0).start()
      out_copy(j0 - 1, 1).wait()
      compute(j0 + 1, 1)
      out_copy(j0 + 1, 1).start()
      return _

    lax.fori_loop(1, n_pairs, do_pair, 0)

    jt = per_w - 1
    ids_copy(jt, 0).wait()
    out_copy(jt - 2, 0).wait()
    compute(jt, 0)
    out_copy(jt, 0).start()
    out_copy(jt - 1, 1).wait()
    out_copy(jt, 0).wait()

  return sc_lookup


def kernel(input_ids, embed_weight, head_weight, head_bias):
  b, t = input_ids.shape
  v, d = embed_weight.shape
  ids_t = input_ids.astype(jnp.int32).T
  lookup = _make_sc_lookup(b, t, v, d)
  out5 = lookup(ids_t, embed_weight, head_weight, head_bias)
  return out5.transpose(2, 4, 0, 1, 3).reshape(b, t, v)


# speedup vs baseline: 1.1593x; 1.1593x over previous
import functools

import jax
import jax.numpy as jnp
from jax import lax
from jax.experimental import pallas as pl
from jax.experimental.pallas import tpu as pltpu
from jax.experimental.pallas import tpu_sc as plsc

_L = 16


def _make_sc_lookup(b: int, t: int, v: int, d: int):
  info = plsc.get_sparse_core_info()
  nw = info.num_cores * info.num_subcores
  assert v == 32 and b % 128 == 0
  nvt = v // 8
  nbt = b // 128
  n_units = t * nvt
  assert n_units % nw == 0
  per_w = n_units // nw

  mesh = plsc.VectorSubcoreMesh(core_axis_name="c", subcore_axis_name="s")

  @functools.partial(
      pl.kernel,
      out_type=jax.ShapeDtypeStruct((t, nvt, nbt, 8, 128), jnp.float32),
      mesh=mesh,
      compiler_params=pltpu.CompilerParams(
          needs_layout_passes=False, use_tc_tiling_on_sc=False
      ),
      scratch_types=[
          pltpu.VMEM((v, d), jnp.float32),
          pltpu.VMEM((v, d), jnp.float32),
          pltpu.VMEM((1, v), jnp.float32),
          pltpu.VMEM((v * v * _L,), jnp.float32),
          pltpu.VMEM((2, b), jnp.int32),
          pltpu.VMEM((2, nbt, 8, 128), jnp.float32),
          pltpu.SemaphoreType.DMA,
          pltpu.SemaphoreType.DMA,
          pltpu.SemaphoreType.DMA,
          pltpu.SemaphoreType.DMA,
      ],
  )
  def sc_lookup(ids_hbm, emb_hbm, head_hbm, bias_hbm, out_hbm,
                emb_v, head_v, bias_v, table_t, ids_vv, out_vv,
                sem_i0, sem_i1, sem_o0, sem_o1):
    cid = lax.axis_index("c")
    sid = lax.axis_index("s")
    wid = sid * info.num_cores + cid

    iota = lax.iota(jnp.int32, _L)

    def build(p, _):
      p16 = pl.multiple_of(p * _L, _L)
      pvec = p16 + iota
      ivec = lax.shift_right_logical(pvec, 5)
      jvec = lax.bitwise_and(pvec, v - 1)
      zvec = jnp.zeros((_L,), jnp.int32)
      acc = plsc.load_gather(bias_v, [zvec, jvec])
      for dd in range(d):
        dvec = jnp.full((_L,), dd, jnp.int32)
        e = plsc.load_gather(emb_v, [ivec, dvec])
        h = plsc.load_gather(head_v, [jvec, dvec])
        acc = acc + e * h
      bvec = lax.shift_left(jvec, 9) + lax.shift_left(ivec, 4)
      for m in range(_L):
        rot = lax.bitwise_and(iota + m, _L - 1)
        plsc.store_scatter(table_t, [bvec + rot], acc)
      return _

    sem_ids = (sem_i0, sem_i1)
    sem_out = (sem_o0, sem_o1)

    def unit_tv(j):
      u = wid * per_w + j
      return u // nvt, lax.rem(u, nvt)

    def ids_copy(j, buf):
      tt, _ = unit_tv(j)
      return pltpu.make_async_copy(
          ids_hbm.at[tt], ids_vv.at[buf], sem_ids[buf])

    ids_copy(0, 0).start()
    pltpu.sync_copy(emb_hbm, emb_v)
    pltpu.sync_copy(head_hbm, head_v)
    pltpu.sync_copy(bias_hbm, bias_v.at[0])
    lax.fori_loop(0, (v * v) // _L, build, 0)

    def out_copy(j, buf):
      tt, vt = unit_tv(j)
      return pltpu.make_async_copy(
          out_vv.at[buf], out_hbm.at[tt, vt], sem_out[buf])

    def both_bufs(sel, fn):
      pl.when(sel == 0)(lambda: fn(0))
      pl.when(sel == 1)(lambda: fn(1))

    def compute(j, bufsel):
      _, vt = unit_tv(j)
      cb512 = (vt * 8) * (v * _L) + jnp.zeros((_L,), jnp.int32)

      @plsc.parallel_loop(0, nbt, 1)
      def bt_body(bt):
        bases = []
        for bi0 in range(8):
          off = pl.multiple_of(bt * 128 + bi0 * _L, _L)
          idvec = ids_vv[bufsel, pl.ds(off, _L)]
          bases.append(
              lax.bitwise_or(lax.shift_left(idvec, 4), iota) + cb512)
        for vi in range(8):
          for bi0 in range(8):
            val = plsc.load_gather(table_t, [bases[bi0] + vi * (v * _L)])
            out_vv[bufsel, bt, vi, pl.ds(bi0 * _L, _L)] = val

    def do_unit(j, _):
      buf = lax.rem(j, 2)
      both_bufs(buf, lambda bb: ids_copy(j, bb).wait())
      pl.when(j + 1 < per_w)(
          lambda: both_bufs(1 - buf, lambda bb: ids_copy(j + 1, bb).start()))
      pl.when(j >= 2)(
          lambda: both_bufs(buf, lambda bb: out_copy(j - 2, bb).wait()))
      compute(j, buf)
      both_bufs(buf, lambda bb: out_copy(j, bb).start())
      return _

    lax.fori_loop(0, per_w, do_unit, 0)
    if per_w >= 2:
      out_copy(per_w - 2, (per_w - 2) % 2).wait()
    out_copy(per_w - 1, (per_w - 1) % 2).wait()

  return sc_lookup


def kernel(input_ids, embed_weight, head_weight, head_bias):
  b, t = input_ids.shape
  v, d = embed_weight.shape
  ids_t = input_ids.astype(jnp.int32).T
  lookup = _make_sc_lookup(b, t, v, d)
  out5 = lookup(ids_t, embed_weight, head_weight, head_bias)
  return out5.transpose(2, 4, 0, 1, 3).reshape(b, t, v)
